# baseline (speedup 1.0000x reference)
import jax
import jax.numpy as jnp
from jax import lax
from jax.experimental import pallas as pl
from jax.experimental.pallas import tpu as pltpu
from jax.experimental.pallas import tpu_sc as plsc

B = 16384
D = 64
NC = 2
NS = 16
NW = NC * NS
BPW = B // NW


def _emb_body(xu_hbm, xp_hbm, xn_hbm, uid_hbm, iid_hbm, out_hbm,
              iu_v, ip_v, in_v, ou_v, op_v, on_v, u_v, p_v, n_v,
              su, sp, sn, swu, swp, swn):
    c = lax.axis_index("c")
    s = lax.axis_index("s")
    wid = s * NC + c
    base = wid * BPW
    pltpu.sync_copy(xu_hbm.at[pl.ds(base, BPW)], iu_v)
    pltpu.sync_copy(xp_hbm.at[pl.ds(base, BPW)], ip_v)
    pltpu.sync_copy(xn_hbm.at[pl.ds(base, BPW)], in_v)
    cu = pltpu.async_copy(uid_hbm.at[iu_v], u_v, su)
    cp = pltpu.async_copy(iid_hbm.at[ip_v], p_v, sp)
    cn = pltpu.async_copy(iid_hbm.at[in_v], n_v, sn)
    base3 = base * 3
    for i in range(BPW // 16):
        v = lax.iota(jnp.int32, 16) * 3 + (base3 + 48 * i)
        ou_v[pl.ds(i * 16, 16)] = v
        op_v[pl.ds(i * 16, 16)] = v + 1
        on_v[pl.ds(i * 16, 16)] = v + 2
    cu.wait()
    wu = pltpu.async_copy(u_v, out_hbm.at[ou_v], swu)
    cp.wait()
    wp = pltpu.async_copy(p_v, out_hbm.at[op_v], swp)
    cn.wait()
    wn = pltpu.async_copy(n_v, out_hbm.at[on_v], swn)
    wu.wait()
    wp.wait()
    wn.wait()


def kernel(x, uid_table, iid_table):
    x = x.astype(jnp.int32)
    xu = x[:, 0]
    xp = x[:, 1]
    xn = x[:, 2]
    mesh = plsc.VectorSubcoreMesh(core_axis_name="c", subcore_axis_name="s")
    k = pl.kernel(
        _emb_body,
        out_type=jax.ShapeDtypeStruct((3 * B, D), jnp.float32),
        mesh=mesh,
        compiler_params=pltpu.CompilerParams(
            use_tc_tiling_on_sc=False,
            has_side_effects=False,
        ),
        scratch_types=[
            pltpu.VMEM((BPW,), jnp.int32),
            pltpu.VMEM((BPW,), jnp.int32),
            pltpu.VMEM((BPW,), jnp.int32),
            pltpu.VMEM((BPW,), jnp.int32),
            pltpu.VMEM((BPW,), jnp.int32),
            pltpu.VMEM((BPW,), jnp.int32),
            pltpu.VMEM((BPW, D), jnp.float32),
            pltpu.VMEM((BPW, D), jnp.float32),
            pltpu.VMEM((BPW, D), jnp.float32),
            pltpu.SemaphoreType.DMA,
            pltpu.SemaphoreType.DMA,
            pltpu.SemaphoreType.DMA,
            pltpu.SemaphoreType.DMA,
            pltpu.SemaphoreType.DMA,
            pltpu.SemaphoreType.DMA,
        ],
    )
    out = k(xu, xp, xn, uid_table, iid_table)
    return out.reshape(B, 3, D)


# speedup vs baseline: 1.5113x; 1.5113x over previous
import jax
import jax.numpy as jnp
from jax import lax
from jax.experimental import pallas as pl
from jax.experimental.pallas import tpu as pltpu
from jax.experimental.pallas import tpu_sc as plsc

B = 16384
D = 64
NC = 2
NS = 16
NW = NC * NS
BPW = B // NW
HB = BPW // 2


def _emb_body(xu_hbm, xp_hbm, xn_hbm, uid_hbm, iid_hbm, out_hbm,
              iu_v, ip_v, in_v, big_v, sem):
    c = lax.axis_index("c")
    s = lax.axis_index("s")
    wid = s * NC + c
    base = wid * BPW
    pltpu.sync_copy(xu_hbm.at[pl.ds(base, BPW)], iu_v)
    pltpu.sync_copy(xp_hbm.at[pl.ds(base, BPW)], ip_v)
    pltpu.sync_copy(xn_hbm.at[pl.ds(base, BPW)], in_v)

    def half(h, carry):
        @plsc.parallel_loop(0, HB // 16, unroll=2)
        def group(g):
            r = h * HB + g * 16
            vu = iu_v[pl.ds(r, 16)]
            vp = ip_v[pl.ds(r, 16)]
            vn = in_v[pl.ds(r, 16)]
            for j in range(16):
                d = 3 * (g * 16 + j)
                pltpu.async_copy(uid_hbm.at[pl.ds(vu[j], 1)],
                                 big_v.at[pl.ds(d, 1)], sem)
                pltpu.async_copy(iid_hbm.at[pl.ds(vp[j], 1)],
                                 big_v.at[pl.ds(d + 1, 1)], sem)
                pltpu.async_copy(iid_hbm.at[pl.ds(vn[j], 1)],
                                 big_v.at[pl.ds(d + 2, 1)], sem)

        pltpu.make_async_copy(uid_hbm.at[pl.ds(0, 3 * HB)], big_v, sem).wait()
        pltpu.sync_copy(big_v, out_hbm.at[pl.ds(3 * (base + h * HB), 3 * HB)])
        return carry

    lax.fori_loop(0, 2, half, 0)


def kernel(x, uid_table, iid_table):
    x = x.astype(jnp.int32)
    xu = x[:, 0]
    xp = x[:, 1]
    xn = x[:, 2]
    mesh = plsc.VectorSubcoreMesh(core_axis_name="c", subcore_axis_name="s")
    k = pl.kernel(
        _emb_body,
        out_type=jax.ShapeDtypeStruct((3 * B, D), jnp.float32),
        mesh=mesh,
        compiler_params=pltpu.CompilerParams(use_tc_tiling_on_sc=True),
        scratch_types=[
            pltpu.VMEM((BPW,), jnp.int32),
            pltpu.VMEM((BPW,), jnp.int32),
            pltpu.VMEM((BPW,), jnp.int32),
            pltpu.VMEM((3 * HB, D), jnp.float32),
            pltpu.SemaphoreType.DMA,
        ],
    )
    out = k(xu, xp, xn, uid_table, iid_table)
    return out.reshape(B, 3, D)


# speedup vs baseline: 1.5174x; 1.0040x over previous
import jax
import jax.numpy as jnp
from jax import lax
from jax.experimental import pallas as pl
from jax.experimental.pallas import tpu as pltpu
from jax.experimental.pallas import tpu_sc as plsc

B = 16384
D = 64
NC = 2
NS = 16
NW = NC * NS
BPW = B // NW
HB = BPW // 2


def _emb_body(xu_hbm, xp_hbm, xn_hbm, uid_hbm, iid_hbm, out_hbm,
              iu_v, ip_v, in_v, big_v, sem):
    c = lax.axis_index("c")
    s = lax.axis_index("s")
    wid = s * NC + c
    base = wid * BPW
    pltpu.sync_copy(xu_hbm.at[pl.ds(base, BPW)], iu_v)
    pltpu.sync_copy(xp_hbm.at[pl.ds(base, BPW)], ip_v)
    pltpu.sync_copy(xn_hbm.at[pl.ds(base, BPW)], in_v)

    def half(h, carry):
        @plsc.parallel_loop(0, HB // 16, unroll=2)
        def group(g):
            r = h * HB + g * 16
            vu = iu_v[pl.ds(r, 16)]
            vp = ip_v[pl.ds(r, 16)]
            vn = in_v[pl.ds(r, 16)]
            for j in range(16):
                d = 3 * (g * 16 + j)
                pltpu.async_copy(uid_hbm.at[pl.ds(vu[j], 1)],
                                 big_v.at[pl.ds(d, 1)], sem)
                pltpu.async_copy(iid_hbm.at[pl.ds(vp[j], 1)],
                                 big_v.at[pl.ds(d + 1, 1)], sem)
                pltpu.async_copy(iid_hbm.at[pl.ds(vn[j], 1)],
                                 big_v.at[pl.ds(d + 2, 1)], sem)

        pltpu.make_async_copy(uid_hbm.at[pl.ds(0, 3 * HB)], big_v, sem).wait()
        pltpu.sync_copy(big_v, out_hbm.at[pl.ds(3 * (base + h * HB), 3 * HB)])
        return carry

    lax.fori_loop(0, 2, half, 0)


def kernel(x, uid_table, iid_table):
    x = x.astype(jnp.int32)
    xu = x[:, 0]
    xp = x[:, 1]
    xn = x[:, 2]
    mesh = plsc.VectorSubcoreMesh(core_axis_name="c", subcore_axis_name="s")
    k = pl.kernel(
        _emb_body,
        out_type=jax.ShapeDtypeStruct((3 * B, D), jnp.float32),
        mesh=mesh,
        compiler_params=pltpu.CompilerParams(
            use_tc_tiling_on_sc=True,
            has_side_effects=False,
        ),
        scratch_types=[
            pltpu.VMEM((BPW,), jnp.int32),
            pltpu.VMEM((BPW,), jnp.int32),
            pltpu.VMEM((BPW,), jnp.int32),
            pltpu.VMEM((3 * HB, D), jnp.float32),
            pltpu.SemaphoreType.DMA,
        ],
    )
    out = k(xu, xp, xn, uid_table, iid_table)
    return out.reshape(B, 3, D)
